# baseline (speedup 1.0000x reference)
import functools

import jax
import jax.numpy as jnp
from jax.experimental import pallas as pl

BETA = 2.0
LOSS_WEIGHT = 1.0


def _qfl_block(pred_ref, label_ref, score_ref, out_ref):
    i = pl.program_id(0)

    p = pred_ref[...]
    lab = label_ref[0]
    sc = score_ref[0]

    B, C = p.shape

    s = jax.nn.sigmoid(p)
    neg = jax.nn.softplus(p) * s * s
    dense_sum = jnp.sum(neg)

    col = jax.lax.broadcasted_iota(jnp.int32, (B, C), 1)
    onehot = col == lab
    pos = (lab >= 0) & (lab < C)

    pred_col = jnp.sum(jnp.where(onehot, p, 0.0), axis=1, keepdims=True)
    neg_col = jnp.sum(jnp.where(onehot, neg, 0.0), axis=1, keepdims=True)

    s_col = jax.nn.sigmoid(pred_col)
    bce = sc * jax.nn.softplus(-pred_col) + (1.0 - sc) * jax.nn.softplus(pred_col)
    diff = jnp.abs(sc - s_col)
    rep = bce * diff * diff

    corr_sum = jnp.sum(jnp.where(pos, rep - neg_col, 0.0))

    @pl.when(i == 0)
    def _init():
        out_ref[...] = jnp.zeros((1, 1), jnp.float32)

    out_ref[...] += jnp.full((1, 1), dense_sum + corr_sum, jnp.float32)


@functools.partial(jax.jit, static_argnames=("block_rows",))
def _qfl(pred, label, score, block_rows=2000):
    N, C = pred.shape
    nb = N // block_rows
    lab3 = label.astype(jnp.int32).reshape(nb, block_rows, 1)
    sc3 = score.reshape(nb, block_rows, 1)

    total = pl.pallas_call(
        _qfl_block,
        grid=(nb,),
        in_specs=[
            pl.BlockSpec((block_rows, C), lambda i: (i, 0)),
            pl.BlockSpec((1, block_rows, 1), lambda i: (i, 0, 0)),
            pl.BlockSpec((1, block_rows, 1), lambda i: (i, 0, 0)),
        ],
        out_specs=pl.BlockSpec((1, 1), lambda i: (0, 0)),
        out_shape=jax.ShapeDtypeStruct((1, 1), jnp.float32),
    )(pred, lab3, sc3)

    return LOSS_WEIGHT * total[0, 0] / N


def kernel(pred, label, score):
    return _qfl(pred, label, score)


# speedup vs baseline: 1.2813x; 1.2813x over previous
import functools

import jax
import jax.numpy as jnp
from jax.experimental import pallas as pl
from jax.experimental.pallas import tpu as pltpu

BETA = 2.0
LOSS_WEIGHT = 1.0


def _qfl_block(pred_ref, label_ref, score_ref, out_ref, acc_ref, acc_corr_ref):
    i = pl.program_id(0)
    nb = pl.num_programs(0)

    @pl.when(i == 0)
    def _init():
        acc_ref[...] = jnp.zeros_like(acc_ref)
        acc_corr_ref[...] = jnp.zeros_like(acc_corr_ref)

    p = pred_ref[...]
    lab = label_ref[0]
    sc = score_ref[0]

    B, C = p.shape

    e = jnp.exp(-p)
    l1 = jnp.log1p(e)
    d = 1.0 / (1.0 + e)
    neg = (p + l1) * (d * d)

    acc_ref[...] += jnp.sum(neg.reshape(B // 8, 8, C), axis=0)

    col = jax.lax.broadcasted_iota(jnp.int32, (B, C), 1)
    onehot = col == lab
    pos = (lab >= 0) & (lab < C)

    pcol = jnp.sum(jnp.where(onehot, p, 0.0), axis=1, keepdims=True)

    ec = jnp.exp(-pcol)
    lc = jnp.log1p(ec)
    dc = 1.0 / (1.0 + ec)
    neg_col = (pcol + lc) * (dc * dc)
    sp = pcol + lc
    bce = sc * (sp - pcol) + (1.0 - sc) * sp
    diff = sc - dc
    rep = bce * diff * diff

    corr = jnp.where(pos, rep - neg_col, 0.0)
    acc_corr_ref[...] += jnp.sum(corr.reshape(B // 8, 8, 1), axis=0)

    @pl.when(i == nb - 1)
    def _fin():
        total = jnp.sum(acc_ref[...]) + jnp.sum(acc_corr_ref[...])
        out_ref[...] = jnp.full((1, 1), total, jnp.float32)


@functools.partial(jax.jit, static_argnames=("block_rows",))
def _qfl(pred, label, score, block_rows=5000):
    N, C = pred.shape
    nb = N // block_rows
    lab3 = label.astype(jnp.int32).reshape(nb, block_rows, 1)
    sc3 = score.reshape(nb, block_rows, 1)

    total = pl.pallas_call(
        _qfl_block,
        grid=(nb,),
        in_specs=[
            pl.BlockSpec((block_rows, C), lambda i: (i, 0)),
            pl.BlockSpec((1, block_rows, 1), lambda i: (i, 0, 0)),
            pl.BlockSpec((1, block_rows, 1), lambda i: (i, 0, 0)),
        ],
        out_specs=pl.BlockSpec((1, 1), lambda i: (0, 0)),
        out_shape=jax.ShapeDtypeStruct((1, 1), jnp.float32),
        scratch_shapes=[
            pltpu.VMEM((8, C), jnp.float32),
            pltpu.VMEM((8, 1), jnp.float32),
        ],
    )(pred, lab3, sc3)

    return LOSS_WEIGHT * total[0, 0] / N


def kernel(pred, label, score):
    return _qfl(pred, label, score)


# speedup vs baseline: 1.5008x; 1.1713x over previous
import functools

import jax
import jax.numpy as jnp
from jax.experimental import pallas as pl
from jax.experimental.pallas import tpu as pltpu

BETA = 2.0
LOSS_WEIGHT = 1.0


def _qfl_block(pred_ref, label_ref, score_ref, out_ref, acc_ref):
    i = pl.program_id(0)
    nb = pl.num_programs(0)

    @pl.when(i == 0)
    def _init():
        acc_ref[...] = jnp.zeros_like(acc_ref)

    p = pred_ref[...]
    lab = label_ref[0]
    sc = score_ref[0]

    B, C = p.shape

    e = jnp.exp(-p)
    sp = p + jnp.log1p(e)
    d = 1.0 / (1.0 + e)
    neg = sp * (d * d)

    t = sc - d
    rep = (sp - sc * p) * (t * t)

    col = jax.lax.broadcasted_iota(jnp.int32, (B, C), 1)
    mask = col == lab

    contrib = jnp.where(mask, rep, neg)
    acc_ref[...] += jnp.sum(contrib.reshape(B // 8, 8, C), axis=0)

    @pl.when(i == nb - 1)
    def _fin():
        out_ref[...] = jnp.full((1, 1), jnp.sum(acc_ref[...]), jnp.float32)


@functools.partial(jax.jit, static_argnames=("block_rows",))
def _qfl(pred, label, score, block_rows=5000):
    N, C = pred.shape
    nb = N // block_rows
    lab3 = label.astype(jnp.int32).reshape(nb, block_rows, 1)
    sc3 = score.reshape(nb, block_rows, 1)

    total = pl.pallas_call(
        _qfl_block,
        grid=(nb,),
        in_specs=[
            pl.BlockSpec((block_rows, C), lambda i: (i, 0)),
            pl.BlockSpec((1, block_rows, 1), lambda i: (i, 0, 0)),
            pl.BlockSpec((1, block_rows, 1), lambda i: (i, 0, 0)),
        ],
        out_specs=pl.BlockSpec((1, 1), lambda i: (0, 0)),
        out_shape=jax.ShapeDtypeStruct((1, 1), jnp.float32),
        scratch_shapes=[pltpu.VMEM((8, C), jnp.float32)],
    )(pred, lab3, sc3)

    return LOSS_WEIGHT * total[0, 0] / N


def kernel(pred, label, score):
    return _qfl(pred, label, score)


# speedup vs baseline: 3.3130x; 2.2075x over previous
import functools

import jax
import jax.numpy as jnp
from jax.experimental import pallas as pl
from jax.experimental.pallas import tpu as pltpu

BETA = 2.0
LOSS_WEIGHT = 1.0


def _qfl_block(pred_ref, label_ref, score_ref, out_ref, acc_ref):
    i = pl.program_id(0)
    nb = pl.num_programs(0)

    @pl.when(i == 0)
    def _init():
        acc_ref[...] = jnp.zeros_like(acc_ref)

    p = pred_ref[...]
    B, C = p.shape
    lab = label_ref[0].reshape(B, 1)
    sc = score_ref[0].reshape(B, 1)

    e = jnp.exp(-p)
    sp = p + jnp.log1p(e)
    d = 1.0 / (1.0 + e)
    neg = sp * (d * d)

    t = sc - d
    rep = (sp - sc * p) * (t * t)

    col = jax.lax.broadcasted_iota(jnp.int32, (B, C), 1)
    mask = col == lab

    contrib = jnp.where(mask, rep, neg)
    acc_ref[...] += jnp.sum(contrib.reshape(B // 8, 8, C), axis=0)

    @pl.when(i == nb - 1)
    def _fin():
        out_ref[...] = jnp.full((1, 1), jnp.sum(acc_ref[...]), jnp.float32)


@functools.partial(jax.jit, static_argnames=("block_rows",))
def _qfl(pred, label, score, block_rows=5000):
    N, C = pred.shape
    nb = N // block_rows
    lab3 = label.astype(jnp.int32).reshape(nb, 1, block_rows)
    sc3 = score.reshape(nb, 1, block_rows)

    total = pl.pallas_call(
        _qfl_block,
        grid=(nb,),
        in_specs=[
            pl.BlockSpec((block_rows, C), lambda i: (i, 0)),
            pl.BlockSpec((1, 1, block_rows), lambda i: (i, 0, 0)),
            pl.BlockSpec((1, 1, block_rows), lambda i: (i, 0, 0)),
        ],
        out_specs=pl.BlockSpec((1, 1), lambda i: (0, 0)),
        out_shape=jax.ShapeDtypeStruct((1, 1), jnp.float32),
        scratch_shapes=[pltpu.VMEM((8, C), jnp.float32)],
    )(pred, lab3, sc3)

    return LOSS_WEIGHT * total[0, 0] / N


def kernel(pred, label, score):
    return _qfl(pred, label, score)


# speedup vs baseline: 3.5487x; 1.0711x over previous
import functools

import jax
import jax.numpy as jnp
from jax.experimental import pallas as pl
from jax.experimental.pallas import tpu as pltpu

BETA = 2.0
LOSS_WEIGHT = 1.0


def _qfl_block(pred_ref, label_ref, score_ref, out_ref, acc_ref):
    i = pl.program_id(0)
    nb = pl.num_programs(0)

    @pl.when(i == 0)
    def _init():
        acc_ref[...] = jnp.zeros_like(acc_ref)

    p = pred_ref[...]
    B, C = p.shape
    lab = label_ref[0].reshape(B, 1)
    sc = score_ref[0].reshape(B, 1)

    e = jnp.exp(-p)
    u = 1.0 + e
    sp = p + jnp.log(u)
    d = 1.0 / u
    neg = sp * (d * d)

    t = sc - d
    rep = (sp - sc * p) * (t * t)

    col = jax.lax.broadcasted_iota(jnp.int32, (B, C), 1)
    mask = col == lab

    contrib = jnp.where(mask, rep, neg)
    acc_ref[...] += jnp.sum(contrib.reshape(B // 8, 8, C), axis=0)

    @pl.when(i == nb - 1)
    def _fin():
        out_ref[...] = jnp.full((1, 1), jnp.sum(acc_ref[...]), jnp.float32)


@functools.partial(jax.jit, static_argnames=("block_rows",))
def _qfl(pred, label, score, block_rows=5000):
    N, C = pred.shape
    nb = N // block_rows
    lab3 = label.astype(jnp.int32).reshape(nb, 1, block_rows)
    sc3 = score.reshape(nb, 1, block_rows)

    total = pl.pallas_call(
        _qfl_block,
        grid=(nb,),
        in_specs=[
            pl.BlockSpec((block_rows, C), lambda i: (i, 0)),
            pl.BlockSpec((1, 1, block_rows), lambda i: (i, 0, 0)),
            pl.BlockSpec((1, 1, block_rows), lambda i: (i, 0, 0)),
        ],
        out_specs=pl.BlockSpec((1, 1), lambda i: (0, 0)),
        out_shape=jax.ShapeDtypeStruct((1, 1), jnp.float32),
        scratch_shapes=[pltpu.VMEM((8, C), jnp.float32)],
    )(pred, lab3, sc3)

    return LOSS_WEIGHT * total[0, 0] / N


def kernel(pred, label, score):
    return _qfl(pred, label, score)
